# baseline (speedup 1.0000x reference)
import functools

import jax
import jax.numpy as jnp
from jax import lax
from jax.experimental import pallas as pl
from jax.experimental.pallas import tpu as pltpu
from jax.experimental.pallas import tpu_sc as plsc

D = 64
CHUNK = 128
N_WORKERS = 32


def _make_kernel(B, n_batch, n_pos):
    per_w = B // N_WORKERS
    n_chunks = per_w // CHUNK
    mesh = plsc.VectorSubcoreMesh(core_axis_name="c", subcore_axis_name="s")

    @functools.partial(
        pl.kernel,
        mesh=mesh,
        out_type=jax.ShapeDtypeStruct(
            (n_pos, D // 8, n_batch // CHUNK, 8, CHUNK), jnp.float32),
        scratch_types=[
            pltpu.VMEM((per_w,), jnp.int32),
            pltpu.VMEM((CHUNK,), jnp.int32),
            pltpu.VMEM((CHUNK,), jnp.int32),
            pltpu.VMEM((CHUNK,), jnp.int32),
            pltpu.VMEM((CHUNK,), jnp.int32),
            pltpu.VMEM((2, CHUNK, 2 * D), jnp.float32),
            pltpu.VMEM((2, D // 8, 8, CHUNK), jnp.float32),
            pltpu.SemaphoreType.DMA,
            pltpu.SemaphoreType.DMA,
            pltpu.SemaphoreType.DMA,
            pltpu.SemaphoreType.DMA,
        ],
        compiler_params=pltpu.CompilerParams(needs_layout_passes=False),
    )
    def k(idx_hbm, w2_hbm, out_hbm, idx_v, i2a, i2b, pca, pcb,
          buf, obuf, gs0, gs1, ws0, ws1):
        num_cores = 2
        wid = lax.axis_index("s") * num_cores + lax.axis_index("c")
        pltpu.sync_copy(idx_hbm.at[wid], idx_v)
        c0 = wid * n_chunks
        idx2 = (i2a, i2b)
        pcol = (pca, pcb)
        gsems = (gs0, gs1)
        wsems = (ws0, ws1)

        def dst_for(cc):
            flat = (c0 + cc) * CHUNK
            j = flat // n_batch
            ct = (flat % n_batch) // CHUNK
            return out_hbm.at[j, :, ct, :, :]

        def prep(cc, b):
            for m in range(CHUNK // 16):
                t = idx_v[pl.ds(cc * CHUNK + 16 * m, 16)]
                idx2[b][pl.ds(16 * m, 16)] = lax.shift_right_logical(t, 1)
                pcol[b][pl.ds(16 * m, 16)] = lax.shift_left(
                    lax.bitwise_and(t, 1), 6)
            pltpu.async_copy(w2_hbm.at[idx2[b]], buf.at[b], gsems[b])

        prep(0, 0)
        rows = [
            lax.iota(jnp.int32, 16) + 16 * m for m in range(CHUNK // 16)
        ]

        def outer(g, carry):
            for b in range(2):
                cc = g * 2 + b
                nb = 1 - b

                @pl.when(cc + 1 < n_chunks)
                def _():
                    prep(cc + 1, nb)

                pltpu.make_async_copy(
                    w2_hbm.at[idx2[b]], buf.at[b], gsems[b]
                ).wait()

                @pl.when(cc >= 2)
                def _():
                    pltpu.make_async_copy(
                        obuf.at[b], dst_for(cc - 2), wsems[b]
                    ).wait()

                pv = [pcol[b][pl.ds(16 * m, 16)] for m in range(CHUNK // 16)]
                for f in range(D):
                    for m in range(CHUNK // 16):
                        val = plsc.load_gather(
                            buf.at[b], [rows[m], pv[m] + f]
                        )
                        obuf[b, f // 8, f % 8, pl.ds(16 * m, 16)] = val

                pltpu.async_copy(obuf.at[b], dst_for(cc), wsems[b])
            return carry

        lax.fori_loop(0, n_chunks // 2, outer, 0)
        pltpu.make_async_copy(obuf.at[0], dst_for(n_chunks - 2), wsems[0]).wait()
        pltpu.make_async_copy(obuf.at[1], dst_for(n_chunks - 1), wsems[1]).wait()

    return k


def kernel(token_ids, weight):
    n_batch, n_pos = token_ids.shape
    B = token_ids.size
    w2 = weight.reshape(weight.shape[0] // 2, 2 * D)
    idx = token_ids.T.reshape(N_WORKERS, B // N_WORKERS)
    out5 = _make_kernel(B, n_batch, n_pos)(idx, w2)
    out = jnp.transpose(out5, (2, 4, 0, 1, 3)).reshape(n_batch, n_pos, D)
    return out


# speedup vs baseline: 1.3347x; 1.3347x over previous
import functools

import jax
import jax.numpy as jnp
from jax import lax
from jax.experimental import pallas as pl
from jax.experimental.pallas import tpu as pltpu
from jax.experimental.pallas import tpu_sc as plsc

D = 64
CHUNK = 128
PAD = 65
N_WORKERS = 32


def _make_kernel(B, n_batch, n_pos):
    per_w = B // N_WORKERS
    n_chunks = per_w // CHUNK
    mesh = plsc.VectorSubcoreMesh(core_axis_name="c", subcore_axis_name="s")

    @functools.partial(
        pl.kernel,
        mesh=mesh,
        out_type=jax.ShapeDtypeStruct(
            (n_pos, D // 8, n_batch // CHUNK, 8, CHUNK), jnp.float32),
        scratch_types=[
            pltpu.VMEM((n_chunks, CHUNK), jnp.int32),
            pltpu.VMEM((2, CHUNK, D), jnp.float32),
            pltpu.VMEM((2, CHUNK * PAD), jnp.float32),
            pltpu.VMEM((2, D // 8, 8, CHUNK), jnp.float32),
            pltpu.SemaphoreType.DMA,
            pltpu.SemaphoreType.DMA,
            pltpu.SemaphoreType.DMA,
            pltpu.SemaphoreType.DMA,
        ],
        compiler_params=pltpu.CompilerParams(
            use_tc_tiling_on_sc=False, needs_layout_passes=False),
    )
    def k(idx_hbm, table_hbm, out_hbm, idx_v, buf, bufp, obuf,
          gs0, gs1, ws0, ws1):
        num_cores = 2
        wid = lax.axis_index("s") * num_cores + lax.axis_index("c")
        pltpu.sync_copy(idx_hbm.at[wid], idx_v)
        c0 = wid * n_chunks
        gsems = (gs0, gs1)
        wsems = (ws0, ws1)

        def dst_for(cc):
            flat = (c0 + cc) * CHUNK
            j = flat // n_batch
            ct = (flat % n_batch) // CHUNK
            return out_hbm.at[j, :, ct, :, :]

        def start_gather(cc, b):
            pltpu.async_copy(
                table_hbm.at[idx_v.at[cc]], buf.at[b], gsems[b])

        start_gather(0, 0)
        rows65 = [
            (lax.iota(jnp.int32, 16) + 16 * m) * PAD
            for m in range(CHUNK // 16)
        ]

        def outer(g, carry):
            for b in range(2):
                cc = g * 2 + b
                nb = 1 - b

                @pl.when(cc + 1 < n_chunks)
                def _():
                    start_gather(cc + 1, nb)

                pltpu.make_async_copy(
                    table_hbm.at[idx_v.at[cc]], buf.at[b], gsems[b]
                ).wait()

                @pl.when(cc >= 2)
                def _():
                    pltpu.make_async_copy(
                        obuf.at[b], dst_for(cc - 2), wsems[b]
                    ).wait()

                for r in range(CHUNK):
                    for c in range(0, D, 16):
                        bufp[b, pl.ds(r * PAD + c, 16)] = (
                            buf[b, r, pl.ds(c, 16)])

                for f in range(D):
                    for m in range(CHUNK // 16):
                        val = plsc.load_gather(
                            bufp.at[b], [rows65[m] + f])
                        obuf[b, f // 8, f % 8, pl.ds(16 * m, 16)] = val

                pltpu.async_copy(obuf.at[b], dst_for(cc), wsems[b])
            return carry

        lax.fori_loop(0, n_chunks // 2, outer, 0)
        pltpu.make_async_copy(obuf.at[0], dst_for(n_chunks - 2), wsems[0]).wait()
        pltpu.make_async_copy(obuf.at[1], dst_for(n_chunks - 1), wsems[1]).wait()

    return k


def kernel(token_ids, weight):
    n_batch, n_pos = token_ids.shape
    B = token_ids.size
    per_w = B // N_WORKERS
    idx = token_ids.T.reshape(N_WORKERS, per_w // CHUNK, CHUNK)
    out5 = _make_kernel(B, n_batch, n_pos)(idx, weight)
    return jnp.transpose(out5, (2, 4, 0, 1, 3)).reshape(n_batch, n_pos, D)


# speedup vs baseline: 1.5712x; 1.1772x over previous
import functools

import jax
import jax.numpy as jnp
from jax import lax
from jax.experimental import pallas as pl
from jax.experimental.pallas import tpu as pltpu
from jax.experimental.pallas import tpu_sc as plsc

D = 64
CHUNK = 512


def _make_kernel(n_workers, n_chunks, per_w, B):
    mesh = plsc.VectorSubcoreMesh(core_axis_name="c", subcore_axis_name="s")

    @functools.partial(
        pl.kernel,
        mesh=mesh,
        out_type=jax.ShapeDtypeStruct((B, D), jnp.float32),
        scratch_types=[
            pltpu.VMEM((n_chunks, CHUNK), jnp.int32),
            pltpu.VMEM((CHUNK, D), jnp.float32),
            pltpu.VMEM((CHUNK, D), jnp.float32),
            pltpu.SemaphoreType.DMA,
            pltpu.SemaphoreType.DMA,
        ],
        compiler_params=pltpu.CompilerParams(use_tc_tiling_on_sc=False),
    )
    def k(idx_hbm, table_hbm, out_hbm, idx_v, buf0, buf1, sem0, sem1):
        num_cores = 2
        wid = lax.axis_index("s") * num_cores + lax.axis_index("c")
        pltpu.sync_copy(idx_hbm.at[wid], idx_v)
        base = wid * per_w
        bufs = (buf0, buf1)
        sems = (sem0, sem1)

        pltpu.async_copy(table_hbm.at[idx_v.at[0]], buf0, sem0)

        def outer(g, carry):
            j0 = g * 2
            for b in range(2):
                j = j0 + b
                nb = 1 - b

                @pl.when(j + 1 < n_chunks)
                def _():
                    pltpu.async_copy(
                        table_hbm.at[idx_v.at[j + 1]], bufs[nb], sems[nb]
                    )

                pltpu.make_async_copy(
                    table_hbm.at[idx_v.at[j]], bufs[b], sems[b]
                ).wait()
                pltpu.sync_copy(
                    bufs[b], out_hbm.at[pl.ds(base + j * CHUNK, CHUNK)]
                )
            return carry

        lax.fori_loop(0, n_chunks // 2, outer, 0)

    return k


def kernel(token_ids, weight):
    B = token_ids.size
    info = plsc.get_sparse_core_info()
    n_workers = info.num_cores * info.num_subcores
    per_w = B // n_workers
    n_chunks = per_w // CHUNK
    idx = token_ids.reshape(n_workers, n_chunks, CHUNK).astype(jnp.int32)
    out = _make_kernel(n_workers, n_chunks, per_w, B)(idx, weight)
    return out.reshape(token_ids.shape + (D,))
